# baseline (speedup 1.0000x reference)
import jax
import jax.numpy as jnp
from jax.experimental import pallas as pl
from jax.experimental.pallas import tpu as pltpu

N_MEM = 50
H, W = 721, 1440
NBINS = N_MEM + 1
BH = 48


def _hist_body(pred_ref, tgt_ref, out_ref, acc_ref):
    i = pl.program_id(0)
    nsteps = pl.num_programs(0)

    @pl.when(i == 0)
    def _init():
        acc_ref[...] = jnp.zeros_like(acc_ref)

    tgt = tgt_ref[...]
    preds = pred_ref[...]
    counts = jnp.sum((preds < tgt[None, :, :]).astype(jnp.int32), axis=0)

    row = i * BH + jax.lax.broadcasted_iota(jnp.int32, (BH, W), 0)
    valid = row < H

    k = jax.lax.broadcasted_iota(jnp.int32, (NBINS, BH, W), 0)
    onehot = ((counts[None, :, :] == k) & valid[None, :, :]).astype(jnp.int32)
    acc_ref[...] += onehot

    @pl.when(i == nsteps - 1)
    def _final():
        out_ref[...] = jnp.sum(jnp.sum(acc_ref[...], axis=1), axis=1,
                               keepdims=True)


@jax.jit
def kernel(predictions, targets):
    nsteps = pl.cdiv(H, BH)
    out = pl.pallas_call(
        _hist_body,
        grid=(nsteps,),
        in_specs=[
            pl.BlockSpec((N_MEM, BH, W), lambda i: (0, i, 0)),
            pl.BlockSpec((BH, W), lambda i: (i, 0)),
        ],
        out_specs=pl.BlockSpec((NBINS, 1), lambda i: (0, 0)),
        out_shape=jax.ShapeDtypeStruct((NBINS, 1), jnp.int32),
        scratch_shapes=[pltpu.VMEM((NBINS, BH, W), jnp.int32)],
    )(predictions, targets)
    return out[:, 0]


# speedup vs baseline: 1.0070x; 1.0070x over previous
import jax
import jax.numpy as jnp
from jax.experimental import pallas as pl
from jax.experimental.pallas import tpu as pltpu

N_MEM = 50
H, W = 721, 1440
NBINS = N_MEM + 1
BH = 32


def _hist_body(pred_ref, tgt_ref, out_ref, acc_ref):
    i = pl.program_id(0)
    nsteps = pl.num_programs(0)

    @pl.when(i == 0)
    def _init():
        acc_ref[...] = jnp.zeros_like(acc_ref)

    tgt = tgt_ref[...]
    preds = pred_ref[...]
    counts = jnp.sum((preds < tgt[None, :, :]).astype(jnp.int32), axis=0)

    row = i * BH + jax.lax.broadcasted_iota(jnp.int32, (BH, W), 0)
    valid = row < H

    k = jax.lax.broadcasted_iota(jnp.int32, (NBINS, BH, W), 0)
    onehot = ((counts[None, :, :] == k) & valid[None, :, :]).astype(jnp.int32)
    acc_ref[...] += onehot

    @pl.when(i == nsteps - 1)
    def _final():
        out_ref[...] = jnp.sum(jnp.sum(acc_ref[...], axis=1), axis=1,
                               keepdims=True)


@jax.jit
def kernel(predictions, targets):
    nsteps = pl.cdiv(H, BH)
    out = pl.pallas_call(
        _hist_body,
        grid=(nsteps,),
        in_specs=[
            pl.BlockSpec((N_MEM, BH, W), lambda i: (0, i, 0)),
            pl.BlockSpec((BH, W), lambda i: (i, 0)),
        ],
        out_specs=pl.BlockSpec((NBINS, 1), lambda i: (0, 0)),
        out_shape=jax.ShapeDtypeStruct((NBINS, 1), jnp.int32),
        scratch_shapes=[pltpu.VMEM((NBINS, BH, W), jnp.int32)],
    )(predictions, targets)
    return out[:, 0]
